# baseline (speedup 1.0000x reference)
import jax
import jax.numpy as jnp
from jax.experimental import pallas as pl
from jax.experimental.pallas import tpu as pltpu


def _soft_quantize_block(z_ref, cb_ref, out_ref):
    z = z_ref[...]
    m = jnp.max(z, axis=-1, keepdims=True)
    e = jnp.exp(z - m)
    s = jnp.sum(e, axis=-1, keepdims=True)
    acc = jnp.dot(e, cb_ref[...], preferred_element_type=jnp.float32)
    out_ref[...] = acc / s


def kernel(z, codebook):
    B, T, E = z.shape
    E2, D = codebook.shape
    n_rows = B * T
    z2 = z.reshape(n_rows, E)
    ROWS = 512
    grid = (n_rows // ROWS,)
    z_q = pl.pallas_call(
        _soft_quantize_block,
        grid=grid,
        in_specs=[
            pl.BlockSpec((ROWS, E), lambda i: (i, 0)),
            pl.BlockSpec((E2, D), lambda i: (0, 0)),
        ],
        out_specs=pl.BlockSpec((ROWS, D), lambda i: (i, 0)),
        out_shape=jax.ShapeDtypeStruct((n_rows, D), z.dtype),
        compiler_params=pltpu.CompilerParams(
            dimension_semantics=("parallel",)),
    )(z2, codebook)
    return (z, z_q.reshape(B, T, D))


# speedup vs baseline: 1.1196x; 1.1196x over previous
import jax
import jax.numpy as jnp
from jax.experimental import pallas as pl
from jax.experimental.pallas import tpu as pltpu


def _soft_quantize_block(z_ref, cb_ref, out_ref):
    z = z_ref[...]
    m = jnp.max(z, axis=-1, keepdims=True)
    e = jnp.exp(z - m)
    s = jnp.sum(e, axis=-1, keepdims=True)
    acc = jnp.dot(e, cb_ref[...], preferred_element_type=jnp.float32)
    out_ref[...] = acc / s


def kernel(z, codebook):
    B, T, E = z.shape
    E2, D = codebook.shape
    n_rows = B * T
    z2 = z.reshape(n_rows, E)
    ROWS = 1152
    grid = (n_rows // ROWS,)
    z_q = pl.pallas_call(
        _soft_quantize_block,
        grid=grid,
        in_specs=[
            pl.BlockSpec((ROWS, E), lambda i: (i, 0)),
            pl.BlockSpec((E2, D), lambda i: (0, 0)),
        ],
        out_specs=pl.BlockSpec((ROWS, D), lambda i: (i, 0)),
        out_shape=jax.ShapeDtypeStruct((n_rows, D), z.dtype),
        compiler_params=pltpu.CompilerParams(
            dimension_semantics=("parallel",)),
    )(z2, codebook)
    return (z, z_q.reshape(B, T, D))


# speedup vs baseline: 1.1436x; 1.0215x over previous
import jax
import jax.numpy as jnp
from jax.experimental import pallas as pl
from jax.experimental.pallas import tpu as pltpu


def _soft_quantize_block(z_ref, cb_ref, out_ref):
    z = z_ref[...]
    m = jnp.max(z, axis=-1, keepdims=True)
    e = jnp.exp(z - m)
    s = jnp.sum(e, axis=-1, keepdims=True)
    acc = jnp.dot(e, cb_ref[...], preferred_element_type=jnp.float32)
    out_ref[...] = acc / s


def kernel(z, codebook):
    B, T, E = z.shape
    E2, D = codebook.shape
    n_rows = B * T
    z2 = z.reshape(n_rows, E)
    ROWS = 2304
    grid = (n_rows // ROWS,)
    z_q = pl.pallas_call(
        _soft_quantize_block,
        grid=grid,
        in_specs=[
            pl.BlockSpec((ROWS, E), lambda i: (i, 0)),
            pl.BlockSpec((E2, D), lambda i: (0, 0)),
        ],
        out_specs=pl.BlockSpec((ROWS, D), lambda i: (i, 0)),
        out_shape=jax.ShapeDtypeStruct((n_rows, D), z.dtype),
        compiler_params=pltpu.CompilerParams(
            dimension_semantics=("parallel",)),
    )(z2, codebook)
    return (z, z_q.reshape(B, T, D))
